# Initial kernel scaffold; baseline (speedup 1.0000x reference)
import jax
import jax.numpy as jnp
from jax.experimental import pallas as pl


def kernel(x, edge_index, batch, W1, b1, W2, b2, W3, b3, W4, b4, Wc, bc):
    raise NotImplementedError("write your pallas kernel here")


# speedup vs baseline: 10.7026x; 10.7026x over previous
import functools

import jax
import jax.numpy as jnp
from jax import lax
from jax.experimental import pallas as pl
from jax.experimental.pallas import tpu as pltpu
from jax.experimental.pallas import tpu_sc as plsc

N = 10000
E = 320000
D = 128
H = 128
G = 32

NC = 2
NS = 16
NW = NC * NS
EPW = E // NW
B = 100
NOPS = EPW // B
CH0 = 640
CHL = N - CH0 * (NS - 1)
ZROWS = 80

TBLK = 1000


def _lin1_body(x_ref, wt_ref, b_ref, o_ref):
    o_ref[...] = (
        jnp.dot(x_ref[...], wt_ref[...], preferred_element_type=jnp.float32)
        + b_ref[...]
    )


def _lin_relu_body(a0_ref, a1_ref, wt_ref, b_ref, o_ref):
    xb = jnp.maximum(a0_ref[0] + a1_ref[0], 0.0)
    o_ref[...] = (
        jnp.dot(xb, wt_ref[...], preferred_element_type=jnp.float32)
        + b_ref[...]
    )


def _pool_body(a0_ref, a1_ref, batch_ref, wct_ref, bc_ref, o_ref):
    h = a0_ref[0] + a1_ref[0]
    bt = batch_ref[...]
    gid = lax.broadcasted_iota(jnp.int32, (G, 1), 0)
    onehot_t = (bt == gid).astype(jnp.float32)
    sums = jnp.dot(onehot_t, h, preferred_element_type=jnp.float32)
    counts = jnp.sum(onehot_t, axis=1, keepdims=True)
    pooled = sums / jnp.maximum(counts, 1.0)
    logits = (
        jnp.dot(pooled, wct_ref[...], preferred_element_type=jnp.float32)
        + bc_ref[...]
    )
    o_ref[...] = 1.0 / (1.0 + jnp.exp(-logits))


def _linear1(x, wt, b2d):
    return pl.pallas_call(
        _lin1_body,
        grid=(N // TBLK,),
        in_specs=[
            pl.BlockSpec((TBLK, D), lambda i: (i, 0)),
            pl.BlockSpec((D, H), lambda i: (0, 0)),
            pl.BlockSpec((1, H), lambda i: (0, 0)),
        ],
        out_specs=pl.BlockSpec((TBLK, H), lambda i: (i, 0)),
        out_shape=jax.ShapeDtypeStruct((N, H), jnp.float32),
    )(x, wt, b2d)


def _linear_relu(a, wt, b2d):
    return pl.pallas_call(
        _lin_relu_body,
        grid=(N // TBLK,),
        in_specs=[
            pl.BlockSpec((1, TBLK, D), lambda i: (0, i, 0)),
            pl.BlockSpec((1, TBLK, D), lambda i: (1, i, 0)),
            pl.BlockSpec((D, H), lambda i: (0, 0)),
            pl.BlockSpec((1, H), lambda i: (0, 0)),
        ],
        out_specs=pl.BlockSpec((TBLK, H), lambda i: (i, 0)),
        out_shape=jax.ShapeDtypeStruct((N, H), jnp.float32),
    )(a, a, wt, b2d)


def _pool_classify(a, batch2d, wct, bc2d):
    return pl.pallas_call(
        _pool_body,
        grid=(1,),
        in_specs=[
            pl.BlockSpec((1, N, H), lambda i: (0, 0, 0)),
            pl.BlockSpec((1, N, H), lambda i: (1, 0, 0)),
            pl.BlockSpec((1, N), lambda i: (0, 0)),
            pl.BlockSpec((H, 1), lambda i: (0, 0)),
            pl.BlockSpec((1, 1), lambda i: (0, 0)),
        ],
        out_specs=pl.BlockSpec((G, 1), lambda i: (0, 0)),
        out_shape=jax.ShapeDtypeStruct((G, 1), jnp.float32),
    )(a, a, batch2d, wct, bc2d)


def _scatter(h, rows3, cols3):
    mesh = plsc.VectorSubcoreMesh(
        core_axis_name="core", subcore_axis_name="subcore"
    )

    @functools.partial(
        pl.kernel,
        out_type=jax.ShapeDtypeStruct((NC, N, H), jnp.float32),
        mesh=mesh,
        scratch_types=[
            pltpu.VMEM((NOPS, B), jnp.int32),
            pltpu.VMEM((NOPS, B), jnp.int32),
            pltpu.VMEM((B, H), jnp.float32),
            pltpu.VMEM((ZROWS, H), jnp.float32),
            pltpu.VMEM_SHARED((N, H), jnp.float32),
            pltpu.SemaphoreType.DMA,
        ],
    )
    def sck(h_hbm, rows_hbm, cols_hbm, out_hbm, rowi, coli, gbuf, zbuf, acc,
            sem):
        c = lax.axis_index("core")
        s = lax.axis_index("subcore")
        wid = s * NC + c

        pltpu.sync_copy(rows_hbm.at[wid], rowi)
        pltpu.sync_copy(cols_hbm.at[wid], coli)

        off = pl.multiple_of(s * CH0, 8)

        @pl.when(c == 0)
        def _():
            @pl.when(s < NS - 1)
            def _():
                pltpu.sync_copy(
                    h_hbm.at[pl.ds(off, CH0)], acc.at[pl.ds(off, CH0)]
                )

            @pl.when(s == NS - 1)
            def _():
                pltpu.sync_copy(
                    h_hbm.at[pl.ds((NS - 1) * CH0, CHL)],
                    acc.at[pl.ds((NS - 1) * CH0, CHL)],
                )

        @pl.when(c != 0)
        def _():
            @pl.loop(0, ZROWS)
            def _(i):
                for j in range(H // 16):
                    zbuf[i, pl.ds(j * 16, 16)] = jnp.zeros((16,), jnp.float32)

            @pl.when(s < NS - 1)
            def _():
                for t in range(CH0 // ZROWS):
                    pltpu.sync_copy(
                        zbuf, acc.at[pl.ds(off + t * ZROWS, ZROWS)]
                    )

            @pl.when(s == NS - 1)
            def _():
                for t in range(CHL // ZROWS):
                    pltpu.sync_copy(
                        zbuf,
                        acc.at[pl.ds((NS - 1) * CH0 + t * ZROWS, ZROWS)],
                    )

        plsc.subcore_barrier()

        @pl.loop(0, NOPS)
        def _(j):
            pltpu.async_copy(h_hbm.at[rowi.at[j]], gbuf, sem).wait()
            pltpu.sync_copy(gbuf, acc.at[coli.at[j]], add=True)

        plsc.subcore_barrier()

        @pl.when(s < NS - 1)
        def _():
            pltpu.sync_copy(
                acc.at[pl.ds(off, CH0)], out_hbm.at[c, pl.ds(off, CH0)]
            )

        @pl.when(s == NS - 1)
        def _():
            pltpu.sync_copy(
                acc.at[pl.ds((NS - 1) * CH0, CHL)],
                out_hbm.at[c, pl.ds((NS - 1) * CH0, CHL)],
            )

    return sck(h, rows3, cols3)


def kernel(x, edge_index, batch, W1, b1, W2, b2, W3, b3, W4, b4, Wc, bc):
    rows3 = edge_index[0].reshape(NW, NOPS, B)
    cols3 = edge_index[1].reshape(NW, NOPS, B)
    batch2d = batch.reshape(1, N)

    h = _linear1(x, W1.T, b1.reshape(1, H))
    a = _scatter(h, rows3, cols3)
    h = _linear_relu(a, W2.T, b2.reshape(1, H))
    a = _scatter(h, rows3, cols3)
    h = _linear_relu(a, W3.T, b3.reshape(1, H))
    a = _scatter(h, rows3, cols3)
    h = _linear_relu(a, W4.T, b4.reshape(1, H))
    a = _scatter(h, rows3, cols3)
    return _pool_classify(a, batch2d, Wc.T, bc.reshape(1, 1))


# speedup vs baseline: 12.6278x; 1.1799x over previous
import functools

import jax
import jax.numpy as jnp
from jax import lax
from jax.experimental import pallas as pl
from jax.experimental.pallas import tpu as pltpu
from jax.experimental.pallas import tpu_sc as plsc

N = 10000
E = 320000
D = 128
H = 128
G = 32

NC = 2
NS = 16
NW = NC * NS
EPW = E // NW
B = 100
NOPS = EPW // B
NHALF = NOPS // 2
NBUF = 2
CH0 = 640
CHL = N - CH0 * (NS - 1)

TBLK = 1000


def _lin1_body(x_ref, wt_ref, b_ref, o_ref):
    o_ref[...] = (
        jnp.dot(x_ref[...], wt_ref[...], preferred_element_type=jnp.float32)
        + b_ref[...]
    )


def _lin_relu_body(a0_ref, a1_ref, h_ref, wt_ref, b_ref, o_ref):
    xb = jnp.maximum(a0_ref[0] + a1_ref[0] - h_ref[...], 0.0)
    o_ref[...] = (
        jnp.dot(xb, wt_ref[...], preferred_element_type=jnp.float32)
        + b_ref[...]
    )


def _pool_body(a0_ref, a1_ref, h_ref, batch_ref, wct_ref, bc_ref, o_ref):
    hh = a0_ref[0] + a1_ref[0] - h_ref[...]
    bt = batch_ref[...]
    gid = lax.broadcasted_iota(jnp.int32, (G, 1), 0)
    onehot_t = (bt == gid).astype(jnp.float32)
    sums = jnp.dot(onehot_t, hh, preferred_element_type=jnp.float32)
    counts = jnp.sum(onehot_t, axis=1, keepdims=True)
    pooled = sums / jnp.maximum(counts, 1.0)
    logits = (
        jnp.dot(pooled, wct_ref[...], preferred_element_type=jnp.float32)
        + bc_ref[...]
    )
    o_ref[...] = 1.0 / (1.0 + jnp.exp(-logits))


def _linear1(x, wt, b2d):
    return pl.pallas_call(
        _lin1_body,
        grid=(N // TBLK,),
        in_specs=[
            pl.BlockSpec((TBLK, D), lambda i: (i, 0)),
            pl.BlockSpec((D, H), lambda i: (0, 0)),
            pl.BlockSpec((1, H), lambda i: (0, 0)),
        ],
        out_specs=pl.BlockSpec((TBLK, H), lambda i: (i, 0)),
        out_shape=jax.ShapeDtypeStruct((N, H), jnp.float32),
    )(x, wt, b2d)


def _linear_relu(a, h, wt, b2d):
    return pl.pallas_call(
        _lin_relu_body,
        grid=(N // TBLK,),
        in_specs=[
            pl.BlockSpec((1, TBLK, D), lambda i: (0, i, 0)),
            pl.BlockSpec((1, TBLK, D), lambda i: (1, i, 0)),
            pl.BlockSpec((TBLK, D), lambda i: (i, 0)),
            pl.BlockSpec((D, H), lambda i: (0, 0)),
            pl.BlockSpec((1, H), lambda i: (0, 0)),
        ],
        out_specs=pl.BlockSpec((TBLK, H), lambda i: (i, 0)),
        out_shape=jax.ShapeDtypeStruct((N, H), jnp.float32),
    )(a, a, h, wt, b2d)


def _pool_classify(a, h, batch2d, wct, bc2d):
    return pl.pallas_call(
        _pool_body,
        grid=(1,),
        in_specs=[
            pl.BlockSpec((1, N, H), lambda i: (0, 0, 0)),
            pl.BlockSpec((1, N, H), lambda i: (1, 0, 0)),
            pl.BlockSpec((N, H), lambda i: (0, 0)),
            pl.BlockSpec((1, N), lambda i: (0, 0)),
            pl.BlockSpec((H, 1), lambda i: (0, 0)),
            pl.BlockSpec((1, 1), lambda i: (0, 0)),
        ],
        out_specs=pl.BlockSpec((G, 1), lambda i: (0, 0)),
        out_shape=jax.ShapeDtypeStruct((G, 1), jnp.float32),
    )(a, a, h, batch2d, wct, bc2d)


def _scatter(h, rows3, cols3):
    mesh = plsc.VectorSubcoreMesh(
        core_axis_name="core", subcore_axis_name="subcore"
    )

    @functools.partial(
        pl.kernel,
        out_type=jax.ShapeDtypeStruct((NC, N, H), jnp.float32),
        mesh=mesh,
        scratch_types=[
            pltpu.VMEM((NHALF, B), jnp.int32),
            pltpu.VMEM((NHALF, B), jnp.int32),
            pltpu.VMEM((B, H), jnp.float32),
            pltpu.VMEM((B, H), jnp.float32),
            pltpu.VMEM_SHARED((N, H), jnp.float32),
            pltpu.SemaphoreType.DMA,
            pltpu.SemaphoreType.DMA,
        ],
    )
    def sck(h_hbm, rows_hbm, cols_hbm, out_hbm, rowi, coli, gbuf0, gbuf1,
            acc, sem0, sem1):
        gbufs = (gbuf0, gbuf1)
        sems = (sem0, sem1)
        c = lax.axis_index("core")
        s = lax.axis_index("subcore")
        wid = s * NC + c

        off = pl.multiple_of(s * CH0, 8)

        @pl.when(s < NS - 1)
        def _():
            pltpu.sync_copy(
                h_hbm.at[pl.ds(off, CH0)], acc.at[pl.ds(off, CH0)]
            )

        @pl.when(s == NS - 1)
        def _():
            pltpu.sync_copy(
                h_hbm.at[pl.ds((NS - 1) * CH0, CHL)],
                acc.at[pl.ds((NS - 1) * CH0, CHL)],
            )

        plsc.subcore_barrier()

        for half in range(2):
            pltpu.sync_copy(rows_hbm.at[wid, half], rowi)
            pltpu.sync_copy(cols_hbm.at[wid, half], coli)

            @pl.loop(0, NHALF, step=NBUF)
            def _(j):
                cps = [
                    pltpu.async_copy(
                        h_hbm.at[rowi.at[j + k]], gbufs[k], sems[k]
                    )
                    for k in range(NBUF)
                ]
                for k in range(NBUF):
                    cps[k].wait()
                    pltpu.sync_copy(
                        gbufs[k], acc.at[coli.at[j + k]], add=True
                    )

        plsc.subcore_barrier()

        @pl.when(s < NS - 1)
        def _():
            pltpu.sync_copy(
                acc.at[pl.ds(off, CH0)], out_hbm.at[c, pl.ds(off, CH0)]
            )

        @pl.when(s == NS - 1)
        def _():
            pltpu.sync_copy(
                acc.at[pl.ds((NS - 1) * CH0, CHL)],
                out_hbm.at[c, pl.ds((NS - 1) * CH0, CHL)],
            )

    return sck(h, rows3, cols3)


def kernel(x, edge_index, batch, W1, b1, W2, b2, W3, b3, W4, b4, Wc, bc):
    rows3 = edge_index[0].reshape(NW, 2, NHALF, B)
    cols3 = edge_index[1].reshape(NW, 2, NHALF, B)
    batch2d = batch.reshape(1, N)

    h = _linear1(x, W1.T, b1.reshape(1, H))
    a = _scatter(h, rows3, cols3)
    h2 = _linear_relu(a, h, W2.T, b2.reshape(1, H))
    a = _scatter(h2, rows3, cols3)
    h3 = _linear_relu(a, h2, W3.T, b3.reshape(1, H))
    a = _scatter(h3, rows3, cols3)
    h4 = _linear_relu(a, h3, W4.T, b4.reshape(1, H))
    a = _scatter(h4, rows3, cols3)
    return _pool_classify(a, h4, batch2d, Wc.T, bc.reshape(1, 1))
